# baseline (speedup 1.0000x reference)
import jax
import jax.numpy as jnp
from jax.experimental import pallas as pl

_B, _C, _H, _W = 8, 32, 64, 64
_K = 1024
_HW = _H * _W
_TW = 4096
_GJ = _HW // _TW


_NCH = 2
_TC_ = _TW // _NCH


def _vq3_kernel(x_ref, c1_ref, c1t_ref, c2_ref, c2t_ref, c3_ref, c3t_ref,
                q1_ref, q2_ref, q3_ref, rec_ref, l1_ref, l2_ref, l3_ref):
    iota = jax.lax.broadcasted_iota(jnp.int32, (_K, _TC_), 0)

    def stage(code_m2_ref, code_t_ref, r):
        code_m2 = code_m2_ref[...]
        code_t = code_t_ref[...]
        cc2 = 0.25 * jnp.sum(code_m2 * code_m2, axis=1, keepdims=True)
        rr2 = jnp.sum(r * r, axis=0, keepdims=True)
        mm2 = jax.lax.dot_general(code_m2, r, (((1,), (0,)), ((), ())),
                                  preferred_element_type=jnp.float32)
        dist = (rr2 + cc2) + mm2
        minv = jnp.min(dist, axis=0, keepdims=True)
        idx = jnp.min(jnp.where(dist == minv, iota, _K), axis=0,
                      keepdims=True)
        onehot = jnp.where(iota == idx, 1.0, 0.0).astype(jnp.float32)
        q = jax.lax.dot_general(code_t, onehot, (((1,), (0,)), ((), ())),
                                preferred_element_type=jnp.float32)
        return q

    sums = []
    for ci in range(_NCH):
        sl = pl.ds(ci * _TC_, _TC_)
        xb = x_ref[0, :, sl]
        q1 = stage(c1_ref, c1t_ref, xb)
        r1 = xb - q1
        q2 = stage(c2_ref, c2t_ref, r1)
        r2 = r1 - q2
        q3 = stage(c3_ref, c3t_ref, r2)
        r3 = r2 - q3
        q1_ref[0, :, sl] = q1
        q2_ref[0, :, sl] = q2
        q3_ref[0, :, sl] = q3
        rec_ref[0, :, sl] = (q1 + q2) + q3
        sums.append((jnp.sum(r1 * r1), jnp.sum(r2 * r2), jnp.sum(r3 * r3)))

    s1 = sums[0][0]
    s2 = sums[0][1]
    s3 = sums[0][2]
    for ci in range(1, _NCH):
        s1 = s1 + sums[ci][0]
        s2 = s2 + sums[ci][1]
        s3 = s3 + sums[ci][2]
    l1_ref[0, 0] = jnp.broadcast_to(s1, (8, 128))
    l2_ref[0, 0] = jnp.broadcast_to(s2, (8, 128))
    l3_ref[0, 0] = jnp.broadcast_to(s3, (8, 128))


def _vq3_call(xf, c1m2, c1t, c2m2, c2t, c3m2, c3t):
    bl = xf.shape[0]
    f32 = jnp.float32

    data_spec = pl.BlockSpec((1, _C, _TW), lambda b, j: (b, 0, j))
    full = pl.BlockSpec((_K, _C), lambda b, j: (0, 0))
    full_t = pl.BlockSpec((_C, _K), lambda b, j: (0, 0))
    loss_spec = pl.BlockSpec((1, 1, 8, 128), lambda b, j: (b, j, 0, 0))

    out_shape = (
        jax.ShapeDtypeStruct((bl, _C, _HW), f32),
        jax.ShapeDtypeStruct((bl, _C, _HW), f32),
        jax.ShapeDtypeStruct((bl, _C, _HW), f32),
        jax.ShapeDtypeStruct((bl, _C, _HW), f32),
        jax.ShapeDtypeStruct((bl, _GJ, 8, 128), f32),
        jax.ShapeDtypeStruct((bl, _GJ, 8, 128), f32),
        jax.ShapeDtypeStruct((bl, _GJ, 8, 128), f32),
    )

    return pl.pallas_call(
        _vq3_kernel,
        grid=(bl, _GJ),
        in_specs=[data_spec, full, full_t, full, full_t, full, full_t],
        out_specs=(data_spec, data_spec, data_spec, data_spec,
                   loss_spec, loss_spec, loss_spec),
        out_shape=out_shape,
    )(xf, c1m2, c1t, c2m2, c2t, c3m2, c3t)


def kernel(x, cur_iter, vq1, vq2, vq3):
    del cur_iter
    xf = x.reshape(_B, _C, _HW)
    args = (xf, -2.0 * vq1, vq1.T, -2.0 * vq2, vq2.T, -2.0 * vq3, vq3.T)

    q1, q2, q3, rec, l1, l2, l3 = _vq3_call(*args)

    scale = 2.0 / (_B * _C * _H * _W)
    loss1 = jnp.sum(l1[:, :, 0, 0]) * scale
    loss2 = jnp.sum(l2[:, :, 0, 0]) * scale
    loss3 = jnp.sum(l3[:, :, 0, 0]) * scale

    shape4 = (_B, _C, _H, _W)
    return (q1.reshape(shape4), q2.reshape(shape4), q3.reshape(shape4),
            loss1, loss2, loss3, rec.reshape(shape4), x, x)


# speedup vs baseline: 1.0068x; 1.0068x over previous
import jax
import jax.numpy as jnp
from jax.experimental import pallas as pl

_B, _C, _H, _W = 8, 32, 64, 64
_K = 1024
_HW = _H * _W
_TW = 4096
_GJ = _HW // _TW


_NCH = 1
_TC_ = _TW // _NCH


def _vq3_kernel(x_ref, c1_ref, c1t_ref, c2_ref, c2t_ref, c3_ref, c3t_ref,
                q1_ref, q2_ref, q3_ref, rec_ref, l1_ref, l2_ref, l3_ref):
    iota = jax.lax.broadcasted_iota(jnp.int32, (_K, _TC_), 0)

    def stage(code_m2_ref, code_t_ref, r):
        code_m2 = code_m2_ref[...]
        code_t = code_t_ref[...]
        cc2 = 0.25 * jnp.sum(code_m2 * code_m2, axis=1, keepdims=True)
        rr2 = jnp.sum(r * r, axis=0, keepdims=True)
        mm2 = jax.lax.dot_general(code_m2, r, (((1,), (0,)), ((), ())),
                                  preferred_element_type=jnp.float32)
        dist = (rr2 + cc2) + mm2
        minv = jnp.min(dist, axis=0, keepdims=True)
        idx = jnp.min(jnp.where(dist == minv, iota, _K), axis=0,
                      keepdims=True)
        onehot = jnp.where(iota == idx, 1.0, 0.0).astype(jnp.float32)
        q = jax.lax.dot_general(code_t, onehot, (((1,), (0,)), ((), ())),
                                preferred_element_type=jnp.float32)
        return q

    sums = []
    for ci in range(_NCH):
        sl = pl.ds(ci * _TC_, _TC_)
        xb = x_ref[0, :, sl]
        q1 = stage(c1_ref, c1t_ref, xb)
        r1 = xb - q1
        q2 = stage(c2_ref, c2t_ref, r1)
        r2 = r1 - q2
        q3 = stage(c3_ref, c3t_ref, r2)
        r3 = r2 - q3
        q1_ref[0, :, sl] = q1
        q2_ref[0, :, sl] = q2
        q3_ref[0, :, sl] = q3
        rec_ref[0, :, sl] = (q1 + q2) + q3
        sums.append((jnp.sum(r1 * r1), jnp.sum(r2 * r2), jnp.sum(r3 * r3)))

    s1 = sums[0][0]
    s2 = sums[0][1]
    s3 = sums[0][2]
    for ci in range(1, _NCH):
        s1 = s1 + sums[ci][0]
        s2 = s2 + sums[ci][1]
        s3 = s3 + sums[ci][2]
    l1_ref[0, 0] = jnp.broadcast_to(s1, (8, 128))
    l2_ref[0, 0] = jnp.broadcast_to(s2, (8, 128))
    l3_ref[0, 0] = jnp.broadcast_to(s3, (8, 128))


def _vq3_call(xf, c1m2, c1t, c2m2, c2t, c3m2, c3t):
    bl = xf.shape[0]
    f32 = jnp.float32

    data_spec = pl.BlockSpec((1, _C, _TW), lambda b, j: (b, 0, j))
    full = pl.BlockSpec((_K, _C), lambda b, j: (0, 0))
    full_t = pl.BlockSpec((_C, _K), lambda b, j: (0, 0))
    loss_spec = pl.BlockSpec((1, 1, 8, 128), lambda b, j: (b, j, 0, 0))

    out_shape = (
        jax.ShapeDtypeStruct((bl, _C, _HW), f32),
        jax.ShapeDtypeStruct((bl, _C, _HW), f32),
        jax.ShapeDtypeStruct((bl, _C, _HW), f32),
        jax.ShapeDtypeStruct((bl, _C, _HW), f32),
        jax.ShapeDtypeStruct((bl, _GJ, 8, 128), f32),
        jax.ShapeDtypeStruct((bl, _GJ, 8, 128), f32),
        jax.ShapeDtypeStruct((bl, _GJ, 8, 128), f32),
    )

    return pl.pallas_call(
        _vq3_kernel,
        grid=(bl, _GJ),
        in_specs=[data_spec, full, full_t, full, full_t, full, full_t],
        out_specs=(data_spec, data_spec, data_spec, data_spec,
                   loss_spec, loss_spec, loss_spec),
        out_shape=out_shape,
    )(xf, c1m2, c1t, c2m2, c2t, c3m2, c3t)


def kernel(x, cur_iter, vq1, vq2, vq3):
    del cur_iter
    xf = x.reshape(_B, _C, _HW)
    args = (xf, -2.0 * vq1, vq1.T, -2.0 * vq2, vq2.T, -2.0 * vq3, vq3.T)

    q1, q2, q3, rec, l1, l2, l3 = _vq3_call(*args)

    scale = 2.0 / (_B * _C * _H * _W)
    loss1 = jnp.sum(l1[:, :, 0, 0]) * scale
    loss2 = jnp.sum(l2[:, :, 0, 0]) * scale
    loss3 = jnp.sum(l3[:, :, 0, 0]) * scale

    shape4 = (_B, _C, _H, _W)
    return (q1.reshape(shape4), q2.reshape(shape4), q3.reshape(shape4),
            loss1, loss2, loss3, rec.reshape(shape4), x, x)
